# baseline (speedup 1.0000x reference)
import functools

import jax
import jax.numpy as jnp
from jax import lax
from jax.experimental import pallas as pl
from jax.experimental.pallas import tpu as pltpu
from jax.experimental.pallas import tpu_sc as plsc

POOL = 16
ROW = 8192
NROWS = 4096
NWORKERS = 32
RPW = NROWS // NWORKERS
GROUPS = ROW // (POOL * 16)


def _row_extrema(in_ref, out_ref, viota):
    def group(g, carry):
        base = g * (POOL * 16)
        in_g = in_ref.at[pl.ds(base, POOL * 16)]
        out_g = out_ref.at[pl.ds(base, POOL * 16)]
        def chain(j0, half):
            idx = viota + j0
            v = plsc.load_gather(in_g, [idx])
            best = jnp.abs(v)
            bval = v
            bidx = idx
            for j in range(j0 + 1, j0 + half):
                idxj = viota + j
                vj = plsc.load_gather(in_g, [idxj])
                aj = jnp.abs(vj)
                gt = aj > best
                best = jnp.maximum(aj, best)
                bval = jnp.where(gt, vj, bval)
                bidx = jnp.where(gt, idxj, bidx)
            return best, bval, bidx

        def merge(na, nb):
            gt = nb[0] > na[0]
            return (jnp.maximum(nb[0], na[0]),
                    jnp.where(gt, nb[1], na[1]),
                    jnp.where(gt, nb[2], na[2]))

        nodes = [chain(j0, 4) for j0 in range(0, POOL, 4)]
        ab = merge(nodes[0], nodes[1])
        cd = merge(nodes[2], nodes[3])
        _, bval, bidx = merge(ab, cd)
        zeros = jnp.zeros((16,), jnp.float32)
        for jj in range(POOL):
            out_g[pl.ds(jj * 16, 16)] = zeros
        plsc.store_scatter(out_g, [bidx], bval)
        return carry

    lax.fori_loop(0, GROUPS, group, 0)


def _make_sc_kernel():
    mesh = plsc.VectorSubcoreMesh(core_axis_name="c", subcore_axis_name="s")

    @functools.partial(
        pl.kernel,
        mesh=mesh,
        out_type=jax.ShapeDtypeStruct((NROWS, ROW), jnp.float32),
        compiler_params=pltpu.CompilerParams(needs_layout_passes=False),
        scratch_types=[
            pltpu.VMEM((ROW,), jnp.float32),
            pltpu.VMEM((ROW,), jnp.float32),
            pltpu.VMEM((ROW,), jnp.float32),
            pltpu.VMEM((ROW,), jnp.float32),
            pltpu.SemaphoreType.DMA,
            pltpu.SemaphoreType.DMA,
            pltpu.SemaphoreType.DMA,
            pltpu.SemaphoreType.DMA,
        ],
    )
    def k(x_hbm, o_hbm, in0, in1, out0, out1, isem0, isem1, osem0, osem1):
        wid = lax.axis_index("s") * 2 + lax.axis_index("c")
        row0 = wid * RPW
        viota = lax.iota(jnp.int32, 16) * 16
        ins = (in0, in1)
        outs = (out0, out1)
        isems = (isem0, isem1)
        osems = (osem0, osem1)

        pltpu.async_copy(x_hbm.at[row0], in0, isem0)
        pltpu.async_copy(x_hbm.at[row0 + 1], in1, isem1)

        def chunk(c, carry):
            for b in range(2):
                r = 2 * c + b
                row = row0 + r
                pltpu.make_async_copy(x_hbm.at[row], ins[b], isems[b]).wait()

                @pl.when(c > 0)
                def _wait_out():
                    pltpu.make_async_copy(outs[b], o_hbm.at[row], osems[b]).wait()

                _row_extrema(ins[b], outs[b], viota)
                pltpu.async_copy(outs[b], o_hbm.at[row], osems[b])

                @pl.when(r + 2 < RPW)
                def _prefetch():
                    pltpu.async_copy(x_hbm.at[row + 2], ins[b], isems[b])
            return carry

        lax.fori_loop(0, RPW // 2, chunk, 0)

        pltpu.make_async_copy(out0, o_hbm.at[row0], osem0).wait()
        pltpu.make_async_copy(out1, o_hbm.at[row0 + 1], osem1).wait()

    return k


_sc_kernel = _make_sc_kernel()


@jax.jit
def kernel(input_):
    B, C, L = input_.shape
    out = _sc_kernel(input_.reshape(B * C, L))
    return out.reshape(B, C, L)


# speedup vs baseline: 1.0536x; 1.0536x over previous
import functools

import jax
import jax.numpy as jnp
from jax import lax
from jax.experimental import pallas as pl
from jax.experimental.pallas import tpu as pltpu
from jax.experimental.pallas import tpu_sc as plsc

POOL = 16
ROW = 8192
NROWS = 4096
NWORKERS = 32
RPW = NROWS // NWORKERS
GROUPS = ROW // (POOL * 16)


def _row_extrema(in_ref, out_ref, viota):
    def group(g, carry):
        base = g * (POOL * 16)
        in_g = in_ref.at[pl.ds(base, POOL * 16)]
        out_g = out_ref.at[pl.ds(base, POOL * 16)]
        def chain(j0, half):
            idx = viota + j0
            v = plsc.load_gather(in_g, [idx])
            best = jnp.abs(v)
            bval = v
            bidx = idx
            for j in range(j0 + 1, j0 + half):
                idxj = viota + j
                vj = plsc.load_gather(in_g, [idxj])
                aj = jnp.abs(vj)
                gt = aj > best
                best = jnp.maximum(aj, best)
                bval = jnp.where(gt, vj, bval)
                bidx = jnp.where(gt, idxj, bidx)
            return best, bval, bidx

        bestA, bvalA, bidxA = chain(0, POOL // 2)
        bestB, bvalB, bidxB = chain(POOL // 2, POOL // 2)
        gtB = bestB > bestA
        bval = jnp.where(gtB, bvalB, bvalA)
        bidx = jnp.where(gtB, bidxB, bidxA)
        zeros = jnp.zeros((16,), jnp.float32)
        for jj in range(POOL):
            out_g[pl.ds(jj * 16, 16)] = zeros
        plsc.store_scatter(out_g, [bidx], bval)
        return carry

    lax.fori_loop(0, GROUPS, group, 0)


def _make_sc_kernel():
    mesh = plsc.VectorSubcoreMesh(core_axis_name="c", subcore_axis_name="s")

    @functools.partial(
        pl.kernel,
        mesh=mesh,
        out_type=jax.ShapeDtypeStruct((NROWS, ROW), jnp.float32),
        compiler_params=pltpu.CompilerParams(needs_layout_passes=False),
        scratch_types=[
            pltpu.VMEM((ROW,), jnp.float32),
            pltpu.VMEM((ROW,), jnp.float32),
            pltpu.VMEM((ROW,), jnp.float32),
            pltpu.VMEM((ROW,), jnp.float32),
            pltpu.SemaphoreType.DMA,
            pltpu.SemaphoreType.DMA,
            pltpu.SemaphoreType.DMA,
            pltpu.SemaphoreType.DMA,
        ],
    )
    def k(x_hbm, o_hbm, in0, in1, out0, out1, isem0, isem1, osem0, osem1):
        wid = lax.axis_index("s") * 2 + lax.axis_index("c")
        row0 = wid * RPW
        viota = lax.iota(jnp.int32, 16) * 16
        ins = (in0, in1)
        outs = (out0, out1)
        isems = (isem0, isem1)
        osems = (osem0, osem1)

        pltpu.async_copy(x_hbm.at[row0], in0, isem0)
        pltpu.async_copy(x_hbm.at[row0 + 1], in1, isem1)

        def chunk(c, carry):
            for b in range(2):
                r = 2 * c + b
                row = row0 + r
                pltpu.make_async_copy(x_hbm.at[row], ins[b], isems[b]).wait()

                @pl.when(c > 0)
                def _wait_out():
                    pltpu.make_async_copy(outs[b], o_hbm.at[row], osems[b]).wait()

                _row_extrema(ins[b], outs[b], viota)
                pltpu.async_copy(outs[b], o_hbm.at[row], osems[b])

                @pl.when(r + 2 < RPW)
                def _prefetch():
                    pltpu.async_copy(x_hbm.at[row + 2], ins[b], isems[b])
            return carry

        lax.fori_loop(0, RPW // 2, chunk, 0)

        pltpu.make_async_copy(out0, o_hbm.at[row0], osem0).wait()
        pltpu.make_async_copy(out1, o_hbm.at[row0 + 1], osem1).wait()

    return k


_sc_kernel = _make_sc_kernel()


@jax.jit
def kernel(input_):
    B, C, L = input_.shape
    out = _sc_kernel(input_.reshape(B * C, L))
    return out.reshape(B, C, L)
